# baseline (speedup 1.0000x reference)
import functools

import jax
import jax.numpy as jnp
from jax import lax
from jax.experimental import pallas as pl
from jax.experimental.pallas import tpu as pltpu
from jax.experimental.pallas import tpu_sc as plsc

BATCH = 16384
DIM = 64
NROWS = 1000000
LANES = 16
NUM_CORES = 2
NUM_SUBCORES = 16
NUM_WORKERS = NUM_CORES * NUM_SUBCORES
BPW = BATCH // NUM_WORKERS
HALFB = BPW // 2
HGROUPS = HALFB // LANES
BLKU = 32768
HBLK = BLKU // 2
BLKU_SH = BLKU.bit_length() - 1
HBLK_SH = HBLK.bit_length() - 1
NBLK = (NROWS + BLKU - 1) // BLKU
NLIN = NBLK * HBLK


def _lin_body(x_ref, eye_ref, o_ref):
    x = x_ref[...]
    xt = lax.dot_general(x, eye_ref[...], (((0,), (0,)), ((), ())),
                         preferred_element_type=jnp.float32)
    o_ref[...] = jnp.concatenate([xt[:HBLK], xt[HBLK:]],
                                 axis=1).astype(jnp.bfloat16)


def _linearize(table_t):
    eye = jnp.eye(DIM, dtype=jnp.float32)
    out = pl.pallas_call(
        _lin_body,
        grid=(NBLK,),
        in_specs=[pl.BlockSpec((DIM, BLKU), lambda i: (0, i)),
                  pl.BlockSpec((DIM, DIM), lambda i: (0, 0))],
        out_specs=pl.BlockSpec((HBLK, 2 * DIM), lambda i: (i, 0)),
        out_shape=jax.ShapeDtypeStruct((NLIN, 2 * DIM), jnp.bfloat16),
    )(table_t, eye)
    return lax.bitcast_convert_type(out.reshape(NLIN, DIM, 2), jnp.int32)


def _row_col(u):
    q = u >> BLKU_SH
    s = u & (BLKU - 1)
    return q * HBLK + (s & (HBLK - 1)), (s >> HBLK_SH)


def _mf_body(uidx_hbm, iidx_hbm, uemb_hbm, iemb_hbm, ubias_hbm, ibias_hbm,
             gbias_hbm, out_hbm,
             uidx_v, iidx_v, urid_v, irid_v, ucol_v, icol_v,
             urows_v, irows_v, ubias_v, ibias_v, gb_v, out_v, sem, bsem):
    wid = lax.axis_index("s") * NUM_CORES + lax.axis_index("c")
    base = wid * BPW

    pltpu.sync_copy(uidx_hbm.at[pl.ds(base, BPW)], uidx_v)
    pltpu.sync_copy(iidx_hbm.at[pl.ds(base, BPW)], iidx_v)
    pltpu.sync_copy(gbias_hbm, gb_v)

    bc1 = pltpu.async_copy(ubias_hbm.at[uidx_v], ubias_v, bsem)
    bc2 = pltpu.async_copy(ibias_hbm.at[iidx_v], ibias_v, bsem)

    def translate(g, carry):
        sl = pl.ds(g * LANES, LANES)
        ur, uc = _row_col(uidx_v[sl])
        ir, ic = _row_col(iidx_v[sl])
        urid_v[sl] = ur
        irid_v[sl] = ir
        ucol_v[sl] = uc * (DIM // 2)
        icol_v[sl] = ic * (DIM // 2)
        return carry

    lax.fori_loop(0, 2 * HGROUPS, translate, 0)

    gb = gb_v[pl.ds(0, LANES)]

    for h in range(2):
        c1 = pltpu.async_copy(uemb_hbm.at[urid_v.at[pl.ds(h * HALFB, HALFB)]],
                              urows_v, sem)
        c2 = pltpu.async_copy(iemb_hbm.at[irid_v.at[pl.ds(h * HALFB, HALFB)]],
                              irows_v, sem)
        c1.wait()
        c2.wait()
        if h == 0:
            bc1.wait()
            bc2.wait()

        def group(g, carry):
            r0 = h * HALFB + g * LANES
            row_ids = g * LANES + lax.iota(jnp.int32, LANES)
            ucol = ucol_v[pl.ds(r0, LANES)]
            icol = icol_v[pl.ds(r0, LANES)]
            acc = gb + ubias_v[pl.ds(r0, LANES)] + ibias_v[pl.ds(r0, LANES)]
            for d2 in range(DIM // 2):
                uw = plsc.load_gather(urows_v, [row_ids, ucol + d2])
                iw = plsc.load_gather(irows_v, [row_ids, icol + d2])
                ua, ub = plsc.unpack(plsc.bitcast(uw, jnp.bfloat16),
                                     format=plsc.PackFormat.INTERLEAVED)
                ia, ib = plsc.unpack(plsc.bitcast(iw, jnp.bfloat16),
                                     format=plsc.PackFormat.INTERLEAVED)
                acc = acc + ua * ia + ub * ib
            out_v[pl.ds(r0, LANES)] = acc
            return carry

        lax.fori_loop(0, HGROUPS, group, 0, unroll=2)

    pltpu.sync_copy(out_v, out_hbm.at[pl.ds(base, BPW)])


@jax.jit
def kernel(user_indices, item_indices, user_embedding, item_embedding,
           user_bias, item_bias, global_bias):
    uemb_lin = _linearize(user_embedding.T)
    iemb_lin = _linearize(item_embedding.T)
    mesh = plsc.VectorSubcoreMesh(core_axis_name="c", subcore_axis_name="s")
    run = functools.partial(
        pl.kernel,
        mesh=mesh,
        compiler_params=pltpu.CompilerParams(needs_layout_passes=False,
                                             use_tc_tiling_on_sc=False),
        out_type=jax.ShapeDtypeStruct((BATCH,), jnp.float32),
        scratch_types=[
            pltpu.VMEM((BPW,), jnp.int32),
            pltpu.VMEM((BPW,), jnp.int32),
            pltpu.VMEM((BPW,), jnp.int32),
            pltpu.VMEM((BPW,), jnp.int32),
            pltpu.VMEM((BPW,), jnp.int32),
            pltpu.VMEM((BPW,), jnp.int32),
            pltpu.VMEM((HALFB, DIM), jnp.int32),
            pltpu.VMEM((HALFB, DIM), jnp.int32),
            pltpu.VMEM((BPW,), jnp.float32),
            pltpu.VMEM((BPW,), jnp.float32),
            pltpu.VMEM((LANES,), jnp.float32),
            pltpu.VMEM((BPW,), jnp.float32),
            pltpu.SemaphoreType.DMA,
            pltpu.SemaphoreType.DMA,
        ],
    )(_mf_body)
    return run(user_indices.astype(jnp.int32), item_indices.astype(jnp.int32),
               uemb_lin, iemb_lin,
               user_bias.reshape(-1), item_bias.reshape(-1),
               jnp.broadcast_to(global_bias, (LANES,)))


# speedup vs baseline: 6.0127x; 6.0127x over previous
import functools

import jax
import jax.numpy as jnp
from jax import lax
from jax.experimental import pallas as pl
from jax.experimental.pallas import tpu as pltpu
from jax.experimental.pallas import tpu_sc as plsc

BATCH = 16384
DIM = 64
NROWS = 1000000
LANES = 16
NUM_CORES = 2
NUM_SUBCORES = 16
NUM_WORKERS = NUM_CORES * NUM_SUBCORES
BPW = BATCH // NUM_WORKERS
HALFB = BPW // 2
HGROUPS = HALFB // LANES
BLKU = 32768
HBLK = BLKU // 2
BLKU_SH = BLKU.bit_length() - 1
HBLK_SH = HBLK.bit_length() - 1
NBLK = (NROWS + BLKU - 1) // BLKU
NLIN = NBLK * HBLK


def _lin_body(x_ref, eye_ref, o_ref):
    x = x_ref[...]
    xt = lax.dot_general(x, eye_ref[...], (((0,), (0,)), ((), ())),
                         preferred_element_type=jnp.float32)
    o_ref[...] = jnp.concatenate([xt[:HBLK], xt[HBLK:]], axis=1)


def _linearize(table_t):
    eye = jnp.eye(DIM, dtype=jnp.float32)
    return pl.pallas_call(
        _lin_body,
        grid=(NBLK,),
        in_specs=[pl.BlockSpec((DIM, BLKU), lambda i: (0, i)),
                  pl.BlockSpec((DIM, DIM), lambda i: (0, 0))],
        out_specs=pl.BlockSpec((HBLK, 2 * DIM), lambda i: (i, 0)),
        out_shape=jax.ShapeDtypeStruct((NLIN, 2 * DIM), jnp.float32),
    )(table_t, eye)


def _row_col(u):
    q = u >> BLKU_SH
    s = u & (BLKU - 1)
    return q * HBLK + (s & (HBLK - 1)), (s >> HBLK_SH)


def _mf_body(uidx_hbm, iidx_hbm, uemb_hbm, iemb_hbm, ubias_hbm, ibias_hbm,
             gbias_hbm, out_hbm,
             uidx_v, iidx_v, urid_v, irid_v, ucol_v, icol_v,
             urows_v, irows_v, ubias_v, ibias_v, gb_v, out_v, sem, bsem):
    wid = lax.axis_index("s") * NUM_CORES + lax.axis_index("c")
    base = wid * BPW

    pltpu.sync_copy(uidx_hbm.at[pl.ds(base, BPW)], uidx_v)
    pltpu.sync_copy(iidx_hbm.at[pl.ds(base, BPW)], iidx_v)
    pltpu.sync_copy(gbias_hbm, gb_v)

    bc1 = pltpu.async_copy(ubias_hbm.at[uidx_v], ubias_v, bsem)
    bc2 = pltpu.async_copy(ibias_hbm.at[iidx_v], ibias_v, bsem)

    def translate(g, carry):
        sl = pl.ds(g * LANES, LANES)
        ur, uc = _row_col(uidx_v[sl])
        ir, ic = _row_col(iidx_v[sl])
        urid_v[sl] = ur
        irid_v[sl] = ir
        ucol_v[sl] = uc * DIM
        icol_v[sl] = ic * DIM
        return carry

    lax.fori_loop(0, 2 * HGROUPS, translate, 0)

    gb = gb_v[pl.ds(0, LANES)]

    for h in range(2):
        c1 = pltpu.async_copy(uemb_hbm.at[urid_v.at[pl.ds(h * HALFB, HALFB)]],
                              urows_v, sem)
        c2 = pltpu.async_copy(iemb_hbm.at[irid_v.at[pl.ds(h * HALFB, HALFB)]],
                              irows_v, sem)
        c1.wait()
        c2.wait()
        if h == 0:
            bc1.wait()
            bc2.wait()

        def group(g, carry):
            r0 = h * HALFB + g * LANES
            row_ids = g * LANES + lax.iota(jnp.int32, LANES)
            ucol = ucol_v[pl.ds(r0, LANES)]
            icol = icol_v[pl.ds(r0, LANES)]
            acc = gb + ubias_v[pl.ds(r0, LANES)] + ibias_v[pl.ds(r0, LANES)]
            for d in range(DIM):
                u_col = plsc.load_gather(urows_v, [row_ids, ucol + d])
                i_col = plsc.load_gather(irows_v, [row_ids, icol + d])
                acc = acc + u_col * i_col
            out_v[pl.ds(r0, LANES)] = acc
            return carry

        lax.fori_loop(0, HGROUPS, group, 0, unroll=2)

    pltpu.sync_copy(out_v, out_hbm.at[pl.ds(base, BPW)])


@jax.jit
def kernel(user_indices, item_indices, user_embedding, item_embedding,
           user_bias, item_bias, global_bias):
    uemb_lin = _linearize(user_embedding.T)
    iemb_lin = _linearize(item_embedding.T)
    mesh = plsc.VectorSubcoreMesh(core_axis_name="c", subcore_axis_name="s")
    run = functools.partial(
        pl.kernel,
        mesh=mesh,
        compiler_params=pltpu.CompilerParams(needs_layout_passes=False,
                                             use_tc_tiling_on_sc=False),
        out_type=jax.ShapeDtypeStruct((BATCH,), jnp.float32),
        scratch_types=[
            pltpu.VMEM((BPW,), jnp.int32),
            pltpu.VMEM((BPW,), jnp.int32),
            pltpu.VMEM((BPW,), jnp.int32),
            pltpu.VMEM((BPW,), jnp.int32),
            pltpu.VMEM((BPW,), jnp.int32),
            pltpu.VMEM((BPW,), jnp.int32),
            pltpu.VMEM((HALFB, 2 * DIM), jnp.float32),
            pltpu.VMEM((HALFB, 2 * DIM), jnp.float32),
            pltpu.VMEM((BPW,), jnp.float32),
            pltpu.VMEM((BPW,), jnp.float32),
            pltpu.VMEM((LANES,), jnp.float32),
            pltpu.VMEM((BPW,), jnp.float32),
            pltpu.SemaphoreType.DMA,
            pltpu.SemaphoreType.DMA,
        ],
    )(_mf_body)
    return run(user_indices.astype(jnp.int32), item_indices.astype(jnp.int32),
               uemb_lin, iemb_lin,
               user_bias.reshape(-1), item_bias.reshape(-1),
               jnp.broadcast_to(global_bias, (LANES,)))
